# Initial kernel scaffold; baseline (speedup 1.0000x reference)
import jax
import jax.numpy as jnp
from jax.experimental import pallas as pl


def kernel(x, lengths, table):
    raise NotImplementedError("write your pallas kernel here")


# speedup vs baseline: 1.2323x; 1.2323x over previous
import functools

import jax
import jax.numpy as jnp
from jax import lax
from jax.experimental import pallas as pl
from jax.experimental.pallas import tpu as pltpu
from jax.experimental.pallas import tpu_sc as plsc

VOCAB = 1000000
WORD_DIM = 128
BATCH = 4096
SEQ = 50

NC = 2
NS = 16
NW = NC * NS

B = BATCH * SEQ
B_PER_W = B // NW
CHUNK = 128
N_CHUNKS = B_PER_W // CHUNK


def _gather_body(x_hbm, table_hbm, out_hbm, idx_v, buf0, buf1, sem0, sem1):
    wid = lax.axis_index("s") * NC + lax.axis_index("c")
    base = wid * B_PER_W
    pltpu.sync_copy(x_hbm.at[wid], idx_v)

    pltpu.async_copy(table_hbm.at[idx_v.at[0]], buf0, sem0)

    def step(j, _):
        @pl.when(j % 2 == 0)
        def _():
            @pl.when(j + 1 < N_CHUNKS)
            def _():
                pltpu.async_copy(table_hbm.at[idx_v.at[j + 1]], buf1, sem1)
            pltpu.make_async_copy(table_hbm.at[idx_v.at[j]], buf0, sem0).wait()
            pltpu.sync_copy(buf0, out_hbm.at[pl.ds(base + j * CHUNK, CHUNK)])

        @pl.when(j % 2 == 1)
        def _():
            @pl.when(j + 1 < N_CHUNKS)
            def _():
                pltpu.async_copy(table_hbm.at[idx_v.at[j + 1]], buf0, sem0)
            pltpu.make_async_copy(table_hbm.at[idx_v.at[j]], buf1, sem1).wait()
            pltpu.sync_copy(buf1, out_hbm.at[pl.ds(base + j * CHUNK, CHUNK)])

        return 0

    lax.fori_loop(0, N_CHUNKS, step, 0)


@jax.jit
def _embed(x_flat, table):
    mesh = plsc.VectorSubcoreMesh(core_axis_name="c", subcore_axis_name="s")
    run = pl.kernel(
        _gather_body,
        out_type=jax.ShapeDtypeStruct((B, WORD_DIM), jnp.float32),
        mesh=mesh,
        scratch_types=[
            pltpu.VMEM((N_CHUNKS, CHUNK), jnp.int32),
            pltpu.VMEM((CHUNK, WORD_DIM), jnp.float32),
            pltpu.VMEM((CHUNK, WORD_DIM), jnp.float32),
            pltpu.SemaphoreType.DMA,
            pltpu.SemaphoreType.DMA,
        ],
    )
    return run(x_flat, table)


def kernel(x, lengths, table):
    x_flat = x.reshape(NW, N_CHUNKS, CHUNK)
    out = _embed(x_flat, table)
    emb = out.reshape(BATCH, SEQ, WORD_DIM)
    return (emb, lengths, emb)
